# Initial kernel scaffold; baseline (speedup 1.0000x reference)
import jax
import jax.numpy as jnp
from jax.experimental import pallas as pl


def kernel(input_ids, token_type_ids, word_table, position_table, token_type_table, ln_gamma, ln_beta):
    raise NotImplementedError("write your pallas kernel here")


# speedup vs baseline: 1.6313x; 1.6313x over previous
import functools
import jax
import jax.numpy as jnp
from jax import lax
from jax.experimental import pallas as pl
from jax.experimental.pallas import tpu as pltpu
from jax.experimental.pallas import tpu_sc as plsc

HID = 1024
EPS = 1e-12

NC = 2
NS = 16
NW = NC * NS


def _sc_gather(word_table, ids_flat, n_rows, chunk):
    b_per_w = n_rows // NW
    n_chunks = b_per_w // chunk
    mesh = plsc.VectorSubcoreMesh(core_axis_name="c", subcore_axis_name="s")

    @functools.partial(
        pl.kernel,
        mesh=mesh,
        out_type=jax.ShapeDtypeStruct((n_rows, HID), jnp.float32),
        scratch_types=[
            pltpu.VMEM((b_per_w,), jnp.int32),
            pltpu.VMEM((chunk, HID), jnp.float32),
            pltpu.SemaphoreType.DMA,
            pltpu.SemaphoreType.DMA,
        ],
    )
    def gather_kernel(table_hbm, idx_hbm, out_hbm, idx_v, rows_v,
                      sem_g, sem_o):
        wid = lax.axis_index("s") * NC + lax.axis_index("c")
        base = wid * b_per_w
        pltpu.sync_copy(idx_hbm.at[pl.ds(base, b_per_w)], idx_v)

        @pl.loop(0, n_chunks)
        def _(i):
            pltpu.async_copy(
                table_hbm.at[idx_v.at[pl.ds(i * chunk, chunk)]], rows_v,
                sem_g).wait()
            pltpu.async_copy(
                rows_v, out_hbm.at[pl.ds(base + i * chunk, chunk)],
                sem_o).wait()

    return gather_kernel(word_table, ids_flat)


def _tc_fuse(words, position_table, tt_f, token_type_table, ln_gamma, ln_beta,
             n_rows, seq_len, block_rows, interpret=False):
    n_blocks = n_rows // block_rows
    s_blocks = seq_len // block_rows

    def body(w_ref, p_ref, ttf_ref, ttab_ref, g_ref, b_ref, o_ref):
        x = w_ref[...] + p_ref[...]
        tt0 = ttab_ref[0, :][None, :]
        dtt = (ttab_ref[1, :] - ttab_ref[0, :])[None, :]
        x = x + tt0 + ttf_ref[...] * dtt
        mu = jnp.mean(x, axis=1, keepdims=True)
        xc = x - mu
        var = jnp.mean(xc * xc, axis=1, keepdims=True)
        y = xc * lax.rsqrt(var + EPS)
        o_ref[...] = y * g_ref[0, :][None, :] + b_ref[0, :][None, :]

    return pl.pallas_call(
        body,
        grid=(n_blocks,),
        in_specs=[
            pl.BlockSpec((block_rows, HID), lambda i: (i, 0)),
            pl.BlockSpec((block_rows, HID), lambda i, _s=s_blocks: (i % _s, 0)),
            pl.BlockSpec((block_rows, 1), lambda i: (i, 0)),
            pl.BlockSpec((2, HID), lambda i: (0, 0)),
            pl.BlockSpec((1, HID), lambda i: (0, 0)),
            pl.BlockSpec((1, HID), lambda i: (0, 0)),
        ],
        out_specs=pl.BlockSpec((block_rows, HID), lambda i: (i, 0)),
        out_shape=jax.ShapeDtypeStruct((n_rows, HID), jnp.float32),
        interpret=interpret,
    )(words, position_table, tt_f, token_type_table, ln_gamma, ln_beta)


def kernel(input_ids, token_type_ids, word_table, position_table,
           token_type_table, ln_gamma, ln_beta):
    B, S = input_ids.shape
    n_rows = B * S
    ids_flat = input_ids.reshape(n_rows).astype(jnp.int32)
    tt_f = token_type_ids.reshape(n_rows, 1).astype(jnp.float32)

    words = _sc_gather(word_table, ids_flat, n_rows, chunk=32)
    out = _tc_fuse(words, position_table, tt_f, token_type_table,
                   ln_gamma.reshape(1, HID), ln_beta.reshape(1, HID),
                   n_rows, S, block_rows=512)
    return out.reshape(B, S, HID)


# speedup vs baseline: 1.7528x; 1.0744x over previous
import functools
import jax
import jax.numpy as jnp
from jax import lax
from jax.experimental import pallas as pl
from jax.experimental.pallas import tpu as pltpu
from jax.experimental.pallas import tpu_sc as plsc

HID = 1024
EPS = 1e-12

NC = 2
NS = 16
NW = NC * NS


def _sc_gather(word_table, ids_flat, n_rows, chunk):
    b_per_w = n_rows // NW
    n_chunks = b_per_w // chunk
    mesh = plsc.VectorSubcoreMesh(core_axis_name="c", subcore_axis_name="s")

    @functools.partial(
        pl.kernel,
        mesh=mesh,
        out_type=jax.ShapeDtypeStruct((n_rows, HID), jnp.float32),
        scratch_types=[
            pltpu.VMEM((b_per_w,), jnp.int32),
            pltpu.VMEM((chunk, HID), jnp.float32),
            pltpu.VMEM((chunk, HID), jnp.float32),
            pltpu.SemaphoreType.DMA,
            pltpu.SemaphoreType.DMA,
            pltpu.SemaphoreType.DMA,
            pltpu.SemaphoreType.DMA,
        ],
    )
    def gather_kernel(table_hbm, idx_hbm, out_hbm, idx_v, rows_a, rows_b,
                      sem_ga, sem_gb, sem_oa, sem_ob):
        wid = lax.axis_index("s") * NC + lax.axis_index("c")
        base = wid * b_per_w
        pltpu.sync_copy(idx_hbm.at[pl.ds(base, b_per_w)], idx_v)

        bufs = (rows_a, rows_b)
        gsems = (sem_ga, sem_gb)
        osems = (sem_oa, sem_ob)

        def gather_in(i):
            return pltpu.make_async_copy(
                table_hbm.at[idx_v.at[pl.ds(i * chunk, chunk)]],
                bufs[i % 2], gsems[i % 2])

        def copy_out(i):
            return pltpu.make_async_copy(
                bufs[i % 2], out_hbm.at[pl.ds(base + i * chunk, chunk)],
                osems[i % 2])

        gather_in(0).start()
        for i in range(n_chunks):
            gather_in(i).wait()
            if i >= 1:
                copy_out(i - 1).wait()
            if i + 1 < n_chunks:
                gather_in(i + 1).start()
            copy_out(i).start()
        copy_out(n_chunks - 1).wait()

    return gather_kernel(word_table, ids_flat)


def _tc_fuse(words, position_table, tt_f, token_type_table, ln_gamma, ln_beta,
             n_rows, seq_len, block_rows, interpret=False):
    n_batch = n_rows // seq_len
    s_blocks = seq_len // block_rows

    def body(w_ref, p_ref, ttf_ref, ttab_ref, g_ref, b_ref, o_ref):
        x = w_ref[...] + p_ref[...]
        tt0 = ttab_ref[0, :][None, :]
        dtt = (ttab_ref[1, :] - ttab_ref[0, :])[None, :]
        x = x + tt0 + ttf_ref[...] * dtt
        mu = jnp.mean(x, axis=1, keepdims=True)
        xc = x - mu
        var = jnp.mean(xc * xc, axis=1, keepdims=True)
        y = xc * lax.rsqrt(var + EPS)
        o_ref[...] = y * g_ref[0, :][None, :] + b_ref[0, :][None, :]

    row_block = lambda s, b, _sb=s_blocks: (b * _sb + s, 0)
    return pl.pallas_call(
        body,
        grid=(s_blocks, n_batch),
        in_specs=[
            pl.BlockSpec((block_rows, HID), row_block),
            pl.BlockSpec((block_rows, HID), lambda s, b: (s, 0)),
            pl.BlockSpec((block_rows, 1), row_block),
            pl.BlockSpec((2, HID), lambda s, b: (0, 0)),
            pl.BlockSpec((1, HID), lambda s, b: (0, 0)),
            pl.BlockSpec((1, HID), lambda s, b: (0, 0)),
        ],
        out_specs=pl.BlockSpec((block_rows, HID), row_block),
        out_shape=jax.ShapeDtypeStruct((n_rows, HID), jnp.float32),
        interpret=interpret,
    )(words, position_table, tt_f, token_type_table, ln_gamma, ln_beta)


def kernel(input_ids, token_type_ids, word_table, position_table,
           token_type_table, ln_gamma, ln_beta):
    B, S = input_ids.shape
    n_rows = B * S
    ids_flat = input_ids.reshape(n_rows).astype(jnp.int32)
    tt_f = token_type_ids.reshape(n_rows, 1).astype(jnp.float32)

    words = _sc_gather(word_table, ids_flat, n_rows, chunk=32)
    out = _tc_fuse(words, position_table, tt_f, token_type_table,
                   ln_gamma.reshape(1, HID), ln_beta.reshape(1, HID),
                   n_rows, S, block_rows=512)
    return out.reshape(B, S, HID)
